# baseline (speedup 1.0000x reference)
import functools

import jax
import jax.numpy as jnp
from jax import lax
from jax.experimental import pallas as pl
from jax.experimental.pallas import tpu as pltpu, tpu_sc as plsc

N = 10000
E0 = 320000
D = 128
DH = D // 2
HID = 1024

NC = 2
NS = 16
NT = NC * NS
CH = 128

E_ALL = E0 + N
NCHT = (-(-E_ALL // (NT * CH)) + 7) // 8 * 8
EPT = NCHT * CH
E_PAD = EPT * NT
GN = 8
NGN = NCHT // GN

CHP = 64
NCHP = EPT // CHP
G = 16
NGRP = NCHP // G
GE = G * CHP

DEG_PAD = 10240
DPT = DEG_PAD // NS

RPT = 624
TAIL = N - NS * RPT

_mesh = plsc.VectorSubcoreMesh(core_axis_name="c", subcore_axis_name="s")
_sc_params = pltpu.CompilerParams(needs_layout_passes=False, use_tc_tiling_on_sc=False)


@functools.partial(
    pl.kernel,
    mesh=_mesh,
    compiler_params=_sc_params,
    out_type=(jax.ShapeDtypeStruct((E_PAD,), jnp.float32),
              jax.ShapeDtypeStruct((N, DH), jnp.float32),
              jax.ShapeDtypeStruct((N, DH), jnp.float32),
              jax.ShapeDtypeStruct((N, DH), jnp.float32),
              jax.ShapeDtypeStruct((N, DH), jnp.float32)),
    scratch_types=[
        pltpu.VMEM_SHARED((DEG_PAD,), jnp.float32),
        pltpu.VMEM_SHARED((N, DH), jnp.float32),
        pltpu.VMEM((3, G, CHP), jnp.int32),
        pltpu.VMEM((G, CHP), jnp.float32),
        pltpu.VMEM((DEG_PAD,), jnp.float32),
        pltpu.VMEM((GE,), jnp.float32),
        pltpu.VMEM((3, CHP, DH), jnp.float32),
        pltpu.SemaphoreType.DMA,
        pltpu.SemaphoreType.DMA,
        pltpu.SemaphoreType.DMA,
        pltpu.SemaphoreType.DMA,
        pltpu.SemaphoreType.DMA,
        pltpu.SemaphoreType.DMA,
    ],
)
def _prop1_kernel(z0, z1, idx3, ew4, zz, norm_out, pa0, pb0, pa1, pb1,
                  deg_s, acc_s, pbuf, ebufF, dis_v, normc, gbuf,
                  sg0, sg1, sg2, ss0, ss1, ss2):
    c = lax.axis_index("c")
    s = lax.axis_index("s")
    w = 2 * s + c
    base_s = s * RPT

    z16 = jnp.zeros((16,), jnp.float32)
    lanes16 = lax.iota(jnp.int32, 16)

    def zb(j, carry):
        normc[pl.ds(j * 16, 16)] = z16
        return carry
    lax.fori_loop(0, DPT // 16, zb, 0)
    pltpu.sync_copy(normc.at[pl.ds(0, DPT)], deg_s.at[pl.ds(s * DPT, DPT)])
    plsc.subcore_barrier()

    for i2 in range(2):
        def dgrp(grp, carry, i2=i2):
            da = pltpu.async_copy(idx3.at[2 * s + i2, grp], pbuf, sg0)
            db = pltpu.async_copy(ew4.at[2 * s + i2, grp], ebufF, sg1)
            da.wait()
            db.wait()
            scs = [pltpu.async_copy(ebufF.at[j], deg_s.at[pbuf.at[1, j]],
                                    ss0, add=True) for j in range(G)]
            for d in scs:
                d.wait()
            return carry
        lax.fori_loop(0, NGN, dgrp, 0)
    plsc.subcore_barrier()

    pltpu.sync_copy(deg_s.at[pl.ds(s * DPT, DPT)], dis_v.at[pl.ds(0, DPT)])

    def rsq(k, carry):
        x = dis_v[pl.ds(k * 16, 16)]
        i = lax.bitcast_convert_type(x, jnp.int32)
        i = 0x5F3759DF - lax.shift_right_arithmetic(i, 1)
        y = lax.bitcast_convert_type(i, jnp.float32)
        for _ in range(3):
            y = y * (1.5 - 0.5 * x * y * y)
        y = jnp.where(x > 0.0, y, 0.0)
        dis_v[pl.ds(k * 16, 16)] = y
        return carry
    lax.fori_loop(0, DPT // 16, rsq, 0)
    pltpu.sync_copy(dis_v.at[pl.ds(0, DPT)], deg_s.at[pl.ds(s * DPT, DPT)])
    plsc.subcore_barrier()
    pltpu.sync_copy(deg_s, dis_v)

    sgs = (sg0, sg1, sg2)
    sss = (ss0, ss1, ss2)

    for p in range(2):
        zp = z0 if p == 0 else z1
        dst0, dst1 = (pa0, pa1) if p == 0 else (pb0, pb1)

        pltpu.sync_copy(zz.at[pl.ds(base_s, RPT)], acc_s.at[pl.ds(base_s, RPT)])

        @pl.when(s == NS - 1)
        def _():
            pltpu.sync_copy(zz.at[pl.ds(NS * RPT, TAIL)],
                            acc_s.at[pl.ds(NS * RPT, TAIL)])
        plsc.subcore_barrier()

        def grp_body(grp, carry, p=p, zp=zp):
            pltpu.sync_copy(idx3.at[w, grp], pbuf)

            def nrm(jj, c2):
                for qq in range(CHP // 16):
                    r16 = pbuf[0, jj, pl.ds(qq * 16, 16)]
                    c16 = pbuf[1, jj, pl.ds(qq * 16, 16)]
                    e16 = lax.bitcast_convert_type(
                        pbuf[2, jj, pl.ds(qq * 16, 16)], jnp.float32)
                    dr = plsc.load_gather(dis_v, [r16])
                    dc = plsc.load_gather(dis_v, [c16])
                    normc[pl.ds(jj * CHP + qq * 16, 16)] = dr * e16 * dc
                return c2
            lax.fori_loop(0, G, nrm, 0)

            if p == 0:
                pltpu.sync_copy(normc,
                                norm_out.at[pl.ds(w * EPT + grp * GE, GE)])

            gd = [None, None, None]
            sd = [None, None, None]
            gd[0] = pltpu.async_copy(zp.at[pbuf.at[0, 0]], gbuf.at[0], sgs[0])
            gd[1] = pltpu.async_copy(zp.at[pbuf.at[0, 1]], gbuf.at[1], sgs[1])
            for k in range(G):
                b = k % 3
                gd[b].wait()

                def scale(g, c3, k=k, b=b):
                    n16 = normc[pl.ds(k * CHP + g * 16, 16)]
                    for bb16 in range(16):
                        nv = n16[bb16]
                        bb = g * 16 + bb16
                        for q in range(DH // 16):
                            gv = gbuf[b, bb, pl.ds(q * 16, 16)]
                            gbuf[b, bb, pl.ds(q * 16, 16)] = gv * nv
                    return c3
                lax.fori_loop(0, CHP // 16, scale, 0)
                sd[b] = pltpu.async_copy(gbuf.at[b], acc_s.at[pbuf.at[1, k]],
                                         sss[b], add=True)
                if k + 2 < G:
                    nb = (k + 2) % 3
                    if sd[nb] is not None:
                        sd[nb].wait()
                        sd[nb] = None
                    gd[nb] = pltpu.async_copy(zp.at[pbuf.at[0, k + 2]],
                                              gbuf.at[nb], sgs[nb])
            for b in range(3):
                if sd[b] is not None:
                    sd[b].wait()
            return carry
        lax.fori_loop(0, NGRP, grp_body, 0)
        plsc.subcore_barrier()

        @pl.when(c == 0)
        def _(dst=dst0):
            pltpu.sync_copy(acc_s.at[pl.ds(base_s, RPT)],
                            dst.at[pl.ds(base_s, RPT)])

            @pl.when(s == NS - 1)
            def _():
                pltpu.sync_copy(acc_s.at[pl.ds(NS * RPT, TAIL)],
                                dst.at[pl.ds(NS * RPT, TAIL)])

        @pl.when(c == 1)
        def _(dst=dst1):
            pltpu.sync_copy(acc_s.at[pl.ds(base_s, RPT)],
                            dst.at[pl.ds(base_s, RPT)])

            @pl.when(s == NS - 1)
            def _():
                pltpu.sync_copy(acc_s.at[pl.ds(NS * RPT, TAIL)],
                                dst.at[pl.ds(NS * RPT, TAIL)])
        plsc.subcore_barrier()


@functools.partial(
    pl.kernel,
    mesh=_mesh,
    compiler_params=_sc_params,
    out_type=(jax.ShapeDtypeStruct((N, DH), jnp.float32),
              jax.ShapeDtypeStruct((N, DH), jnp.float32),
              jax.ShapeDtypeStruct((N, DH), jnp.float32),
              jax.ShapeDtypeStruct((N, DH), jnp.float32)),
    scratch_types=[
        pltpu.VMEM_SHARED((N, DH), jnp.float32),
        pltpu.VMEM((3, G, CHP), jnp.int32),
        pltpu.VMEM((3, CHP, DH), jnp.float32),
        pltpu.SemaphoreType.DMA,
        pltpu.SemaphoreType.DMA,
        pltpu.SemaphoreType.DMA,
        pltpu.SemaphoreType.DMA,
        pltpu.SemaphoreType.DMA,
        pltpu.SemaphoreType.DMA,
    ],
)
def _prop_kernel(z0, z1, packed, zz, pa0, pb0, pa1, pb1,
                 acc_s, pbuf, gbuf,
                 sg0, sg1, sg2, ss0, ss1, ss2):
    c = lax.axis_index("c")
    s = lax.axis_index("s")
    w = 2 * s + c
    base_s = s * RPT

    for p in range(2):
        zp = z0 if p == 0 else z1
        dst0, dst1 = (pa0, pa1) if p == 0 else (pb0, pb1)

        pltpu.sync_copy(zz.at[pl.ds(base_s, RPT)], acc_s.at[pl.ds(base_s, RPT)])

        @pl.when(s == NS - 1)
        def _():
            pltpu.sync_copy(zz.at[pl.ds(NS * RPT, TAIL)],
                            acc_s.at[pl.ds(NS * RPT, TAIL)])
        plsc.subcore_barrier()

        sgs = (sg0, sg1, sg2)
        sss = (ss0, ss1, ss2)

        def grp_body(grp, carry):
            pltpu.sync_copy(packed.at[w].at[grp], pbuf)

            gd = [None, None, None]
            sd = [None, None, None]
            gd[0] = pltpu.async_copy(zp.at[pbuf.at[0, 0]], gbuf.at[0], sgs[0])
            gd[1] = pltpu.async_copy(zp.at[pbuf.at[0, 1]], gbuf.at[1], sgs[1])
            for k in range(G):
                b = k % 3
                gd[b].wait()

                def scale(g, c3, k=k, b=b):
                    nbits = pbuf[2, k, pl.ds(g * 16, 16)]
                    n16 = lax.bitcast_convert_type(nbits, jnp.float32)
                    for bb16 in range(16):
                        nv = n16[bb16]
                        bb = g * 16 + bb16
                        for q in range(DH // 16):
                            gv = gbuf[b, bb, pl.ds(q * 16, 16)]
                            gbuf[b, bb, pl.ds(q * 16, 16)] = gv * nv
                    return c3
                lax.fori_loop(0, CHP // 16, scale, 0)
                sd[b] = pltpu.async_copy(gbuf.at[b], acc_s.at[pbuf.at[1, k]],
                                         sss[b], add=True)
                if k + 2 < G:
                    nb = (k + 2) % 3
                    if sd[nb] is not None:
                        sd[nb].wait()
                        sd[nb] = None
                    gd[nb] = pltpu.async_copy(zp.at[pbuf.at[0, k + 2]],
                                              gbuf.at[nb], sgs[nb])
            for b in range(3):
                if sd[b] is not None:
                    sd[b].wait()
            return carry
        lax.fori_loop(0, NGRP, grp_body, 0)
        plsc.subcore_barrier()

        @pl.when(c == 0)
        def _(dst=dst0):
            pltpu.sync_copy(acc_s.at[pl.ds(base_s, RPT)],
                            dst.at[pl.ds(base_s, RPT)])

            @pl.when(s == NS - 1)
            def _():
                pltpu.sync_copy(acc_s.at[pl.ds(NS * RPT, TAIL)],
                                dst.at[pl.ds(NS * RPT, TAIL)])

        @pl.when(c == 1)
        def _(dst=dst1):
            pltpu.sync_copy(acc_s.at[pl.ds(base_s, RPT)],
                            dst.at[pl.ds(base_s, RPT)])

            @pl.when(s == NS - 1)
            def _():
                pltpu.sync_copy(acc_s.at[pl.ds(NS * RPT, TAIL)],
                                dst.at[pl.ds(NS * RPT, TAIL)])
        plsc.subcore_barrier()


RB = 1000


def _mlp_body(pa0_ref, pb0_ref, pa1_ref, pb1_ref,
              w1a_ref, w1b_ref, b1_ref, w2a_ref, w2b_ref,
              hwa_ref, hwb_ref):
    pa = pa0_ref[...] + pa1_ref[...]
    pb = pb0_ref[...] + pb1_ref[...]
    h = jnp.dot(pa, w1a_ref[...], preferred_element_type=jnp.float32)
    h = h + jnp.dot(pb, w1b_ref[...], preferred_element_type=jnp.float32)
    h = jnp.maximum(h + b1_ref[...], 0.0)
    hwa_ref[...] = jnp.dot(h, w2a_ref[...], preferred_element_type=jnp.float32)
    hwb_ref[...] = jnp.dot(h, w2b_ref[...], preferred_element_type=jnp.float32)


_mlp = pl.pallas_call(
    _mlp_body,
    grid=(N // RB,),
    in_specs=[
        pl.BlockSpec((RB, DH), lambda i: (i, 0)),
        pl.BlockSpec((RB, DH), lambda i: (i, 0)),
        pl.BlockSpec((RB, DH), lambda i: (i, 0)),
        pl.BlockSpec((RB, DH), lambda i: (i, 0)),
        pl.BlockSpec((DH, HID), lambda i: (0, 0)),
        pl.BlockSpec((DH, HID), lambda i: (0, 0)),
        pl.BlockSpec((1, HID), lambda i: (0, 0)),
        pl.BlockSpec((HID, DH), lambda i: (0, 0)),
        pl.BlockSpec((HID, DH), lambda i: (0, 0)),
    ],
    out_specs=(pl.BlockSpec((RB, DH), lambda i: (i, 0)),
               pl.BlockSpec((RB, DH), lambda i: (i, 0))),
    out_shape=(jax.ShapeDtypeStruct((N, DH), jnp.float32),
               jax.ShapeDtypeStruct((N, DH), jnp.float32)),
)


def _add_body(qa0_ref, qb0_ref, qa1_ref, qb1_ref, b2_ref, out_ref):
    qa = qa0_ref[...] + qa1_ref[...]
    qb = qb0_ref[...] + qb1_ref[...]
    out_ref[...] = jnp.concatenate([qa, qb], axis=1) + b2_ref[...]


_add = pl.pallas_call(
    _add_body,
    grid=(N // RB,),
    in_specs=[
        pl.BlockSpec((RB, DH), lambda i: (i, 0)),
        pl.BlockSpec((RB, DH), lambda i: (i, 0)),
        pl.BlockSpec((RB, DH), lambda i: (i, 0)),
        pl.BlockSpec((RB, DH), lambda i: (i, 0)),
        pl.BlockSpec((1, D), lambda i: (0, 0)),
    ],
    out_specs=pl.BlockSpec((RB, D), lambda i: (i, 0)),
    out_shape=jax.ShapeDtypeStruct((N, D), jnp.float32),
)


def kernel(x, edge_index, edge_attr, W1, b1, W2, b2):
    pad = E_PAD - E_ALL
    ar = jnp.arange(N, dtype=jnp.int32)
    pi = jnp.arange(pad, dtype=jnp.int32) % N
    row_f = jnp.concatenate([edge_index[0], ar, pi])
    col_f = jnp.concatenate([edge_index[1], ar, pi])
    ew_f = jnp.concatenate(
        [edge_attr, jnp.ones((N,), jnp.float32), jnp.zeros((pad,), jnp.float32)]
    )
    ew_bits = lax.bitcast_convert_type(ew_f, jnp.int32)
    idx3 = jnp.stack(
        [row_f.reshape(NT, NGRP, G, CHP),
         col_f.reshape(NT, NGRP, G, CHP),
         ew_bits.reshape(NT, NGRP, G, CHP)], axis=2)
    ew4 = ew_f.reshape(NT, NGRP, G, CHP)

    x0 = x[:, :DH]
    x1 = x[:, DH:]
    zz = jnp.zeros((N, DH), jnp.float32)

    norm, pa0, pb0, pa1, pb1 = _prop1_kernel(x0, x1, idx3, ew4, zz)
    hwa, hwb = _mlp(pa0, pb0, pa1, pb1,
                    W1[:DH], W1[DH:], b1.reshape(1, HID),
                    W2[:, :DH], W2[:, DH:])

    nbits = lax.bitcast_convert_type(norm, jnp.int32)
    packed = jnp.stack(
        [row_f.reshape(NT, NGRP, G, CHP),
         col_f.reshape(NT, NGRP, G, CHP),
         nbits.reshape(NT, NGRP, G, CHP)], axis=2)

    qa0, qb0, qa1, qb1 = _prop_kernel(hwa, hwb, packed, zz)
    return _add(qa0, qb0, qa1, qb1, b2.reshape(1, D))


# speedup vs baseline: 1.0528x; 1.0528x over previous
import functools

import jax
import jax.numpy as jnp
from jax import lax
from jax.experimental import pallas as pl
from jax.experimental.pallas import tpu as pltpu, tpu_sc as plsc

N = 10000
E0 = 320000
D = 128
DH = D // 2
HID = 1024

NC = 2
NS = 16
NT = NC * NS
CH = 128

E_ALL = E0 + N
NCHT = (-(-E_ALL // (NT * CH)) + 7) // 8 * 8
EPT = NCHT * CH
E_PAD = EPT * NT
GN = 8
NGN = NCHT // GN

CHP = 64
NCHP = EPT // CHP
G = 16
NGRP = NCHP // G
GE = G * CHP

DEG_PAD = 10240
DPT = DEG_PAD // NS

RPT = 624
TAIL = N - NS * RPT

_mesh = plsc.VectorSubcoreMesh(core_axis_name="c", subcore_axis_name="s")
_sc_params = pltpu.CompilerParams(needs_layout_passes=False, use_tc_tiling_on_sc=False)


@functools.partial(
    pl.kernel,
    mesh=_mesh,
    compiler_params=_sc_params,
    out_type=jax.ShapeDtypeStruct((E_PAD,), jnp.float32),
    scratch_types=[
        pltpu.VMEM_SHARED((DEG_PAD,), jnp.float32),
        pltpu.VMEM_SHARED((DEG_PAD,), jnp.float32),
        pltpu.VMEM((GN, CH), jnp.int32),
        pltpu.VMEM((GN, CH), jnp.float32),
        pltpu.VMEM((GN, CH), jnp.int32),
        pltpu.VMEM((DEG_PAD,), jnp.float32),
        pltpu.VMEM((GN * CH,), jnp.float32),
        pltpu.SemaphoreType.DMA,
        pltpu.SemaphoreType.DMA,
    ],
)
def _norm_kernel(row3d, col3d, ew3d, norm_out,
                 deg_s, dis_s, cbufc, ebufc, rbufc, dis_v, normc, sma, smb):
    c = lax.axis_index("c")
    s = lax.axis_index("s")
    w = 2 * s + c

    z16 = jnp.zeros((16,), jnp.float32)

    def zb(j, carry):
        normc[pl.ds(j * 16, 16)] = z16
        return carry
    lax.fori_loop(0, DPT // 16, zb, 0)
    pltpu.sync_copy(normc.at[pl.ds(0, DPT)], deg_s.at[pl.ds(s * DPT, DPT)])
    plsc.subcore_barrier()

    for i2 in range(2):
        def dgrp(grp, carry, i2=i2):
            da = pltpu.async_copy(
                col3d.at[2 * s + i2].at[pl.ds(grp * GN, GN)], cbufc, sma)
            db = pltpu.async_copy(
                ew3d.at[2 * s + i2].at[pl.ds(grp * GN, GN)], ebufc, smb)
            da.wait()
            db.wait()
            scs = [pltpu.async_copy(ebufc.at[j], deg_s.at[cbufc.at[j]], sma,
                                    add=True) for j in range(GN)]
            for d in scs:
                d.wait()
            return carry
        lax.fori_loop(0, NGN, dgrp, 0)
    plsc.subcore_barrier()

    pltpu.sync_copy(deg_s.at[pl.ds(s * DPT, DPT)], dis_v.at[pl.ds(0, DPT)])

    def rsq(k, carry):
        x = dis_v[pl.ds(k * 16, 16)]
        i = lax.bitcast_convert_type(x, jnp.int32)
        i = 0x5F3759DF - lax.shift_right_arithmetic(i, 1)
        y = lax.bitcast_convert_type(i, jnp.float32)
        for _ in range(3):
            y = y * (1.5 - 0.5 * x * y * y)
        y = jnp.where(x > 0.0, y, 0.0)
        dis_v[pl.ds(k * 16, 16)] = y
        return carry
    lax.fori_loop(0, DPT // 16, rsq, 0)
    pltpu.sync_copy(dis_v.at[pl.ds(0, DPT)], dis_s.at[pl.ds(s * DPT, DPT)])
    plsc.subcore_barrier()
    pltpu.sync_copy(dis_s, dis_v)

    def ngrp(grp, carry):
        pltpu.sync_copy(row3d.at[w].at[pl.ds(grp * GN, GN)], rbufc)
        pltpu.sync_copy(col3d.at[w].at[pl.ds(grp * GN, GN)], cbufc)
        pltpu.sync_copy(ew3d.at[w].at[pl.ds(grp * GN, GN)], ebufc)

        def nrm(j, c2):
            for q in range(8):
                r16 = rbufc[j, pl.ds(q * 16, 16)]
                c16 = cbufc[j, pl.ds(q * 16, 16)]
                e16 = ebufc[j, pl.ds(q * 16, 16)]
                dr = plsc.load_gather(dis_v, [r16])
                dc = plsc.load_gather(dis_v, [c16])
                normc[pl.ds(j * CH + q * 16, 16)] = dr * e16 * dc
            return c2
        lax.fori_loop(0, GN, nrm, 0)
        pltpu.sync_copy(normc,
                        norm_out.at[pl.ds(w * EPT + grp * (GN * CH), GN * CH)])
        return carry
    lax.fori_loop(0, NGN, ngrp, 0)


@functools.partial(
    pl.kernel,
    mesh=_mesh,
    compiler_params=_sc_params,
    out_type=(jax.ShapeDtypeStruct((N, DH), jnp.float32),
              jax.ShapeDtypeStruct((N, DH), jnp.float32),
              jax.ShapeDtypeStruct((N, DH), jnp.float32),
              jax.ShapeDtypeStruct((N, DH), jnp.float32)),
    scratch_types=[
        pltpu.VMEM_SHARED((N, DH), jnp.float32),
        pltpu.VMEM((3, G, CHP), jnp.int32),
        pltpu.VMEM((2, CHP, DH), jnp.bfloat16),
        pltpu.VMEM((2, CHP, DH), jnp.float32),
        pltpu.SemaphoreType.DMA,
        pltpu.SemaphoreType.DMA,
        pltpu.SemaphoreType.DMA,
        pltpu.SemaphoreType.DMA,
    ],
)
def _prop_kernel(z0, z1, packed, zz, pa0, pb0, pa1, pb1,
                 acc_s, pbuf, gbuf, gsc,
                 sg0, sg1, ss0, ss1):
    c = lax.axis_index("c")
    s = lax.axis_index("s")
    w = 2 * s + c
    base_s = s * RPT

    for p in range(2):
        zp = z0 if p == 0 else z1
        dst0, dst1 = (pa0, pa1) if p == 0 else (pb0, pb1)

        pltpu.sync_copy(zz.at[pl.ds(base_s, RPT)], acc_s.at[pl.ds(base_s, RPT)])

        @pl.when(s == NS - 1)
        def _():
            pltpu.sync_copy(zz.at[pl.ds(NS * RPT, TAIL)],
                            acc_s.at[pl.ds(NS * RPT, TAIL)])
        plsc.subcore_barrier()

        sgs = (sg0, sg1)
        sss = (ss0, ss1)

        def grp_body(grp, carry):
            pltpu.sync_copy(packed.at[w].at[grp], pbuf)

            gd = [None, None]
            sd = [None, None]
            gd[0] = pltpu.async_copy(zp.at[pbuf.at[0, 0]], gbuf.at[0], sgs[0])
            gd[1] = pltpu.async_copy(zp.at[pbuf.at[0, 1]], gbuf.at[1], sgs[1])
            for k in range(G):
                b = k % 2
                gd[b].wait()
                if sd[b] is not None:
                    sd[b].wait()
                    sd[b] = None

                def scale(g, c3, k=k, b=b):
                    nbits = pbuf[2, k, pl.ds(g * 16, 16)]
                    n16 = lax.bitcast_convert_type(nbits, jnp.float32)
                    for bb16 in range(16):
                        nv = n16[bb16]
                        bb = g * 16 + bb16
                        for q2 in range(DH // 32):
                            v32 = gbuf[b, bb, pl.ds(q2 * 32, 32)]
                            ev, od = plsc.unpack(
                                v32, format=plsc.PackFormat.INTERLEAVED)
                            gsc[b, bb, pl.ds(q2 * 32, 16)] = ev * nv
                            gsc[b, bb, pl.ds(q2 * 32 + 16, 16)] = od * nv
                    return c3
                lax.fori_loop(0, CHP // 16, scale, 0)
                sd[b] = pltpu.async_copy(gsc.at[b], acc_s.at[pbuf.at[1, k]],
                                         sss[b], add=True)
                if k + 2 < G:
                    gd[b] = pltpu.async_copy(zp.at[pbuf.at[0, k + 2]],
                                             gbuf.at[b], sgs[b])
            for b in range(2):
                if sd[b] is not None:
                    sd[b].wait()
            return carry
        lax.fori_loop(0, NGRP, grp_body, 0)
        plsc.subcore_barrier()

        @pl.when(c == 0)
        def _(dst=dst0):
            pltpu.sync_copy(acc_s.at[pl.ds(base_s, RPT)],
                            dst.at[pl.ds(base_s, RPT)])

            @pl.when(s == NS - 1)
            def _():
                pltpu.sync_copy(acc_s.at[pl.ds(NS * RPT, TAIL)],
                                dst.at[pl.ds(NS * RPT, TAIL)])

        @pl.when(c == 1)
        def _(dst=dst1):
            pltpu.sync_copy(acc_s.at[pl.ds(base_s, RPT)],
                            dst.at[pl.ds(base_s, RPT)])

            @pl.when(s == NS - 1)
            def _():
                pltpu.sync_copy(acc_s.at[pl.ds(NS * RPT, TAIL)],
                                dst.at[pl.ds(NS * RPT, TAIL)])
        plsc.subcore_barrier()


RB = 1000


def _mlp_body(pa0_ref, pb0_ref, pa1_ref, pb1_ref,
              w1a_ref, w1b_ref, b1_ref, w2a_ref, w2b_ref,
              hwa_ref, hwb_ref):
    pa = pa0_ref[...] + pa1_ref[...]
    pb = pb0_ref[...] + pb1_ref[...]
    h = jnp.dot(pa, w1a_ref[...], preferred_element_type=jnp.float32)
    h = h + jnp.dot(pb, w1b_ref[...], preferred_element_type=jnp.float32)
    h = jnp.maximum(h + b1_ref[...], 0.0)
    hwa_ref[...] = jnp.dot(
        h, w2a_ref[...], preferred_element_type=jnp.float32
    ).astype(jnp.bfloat16)
    hwb_ref[...] = jnp.dot(
        h, w2b_ref[...], preferred_element_type=jnp.float32
    ).astype(jnp.bfloat16)


_mlp = pl.pallas_call(
    _mlp_body,
    grid=(N // RB,),
    in_specs=[
        pl.BlockSpec((RB, DH), lambda i: (i, 0)),
        pl.BlockSpec((RB, DH), lambda i: (i, 0)),
        pl.BlockSpec((RB, DH), lambda i: (i, 0)),
        pl.BlockSpec((RB, DH), lambda i: (i, 0)),
        pl.BlockSpec((DH, HID), lambda i: (0, 0)),
        pl.BlockSpec((DH, HID), lambda i: (0, 0)),
        pl.BlockSpec((1, HID), lambda i: (0, 0)),
        pl.BlockSpec((HID, DH), lambda i: (0, 0)),
        pl.BlockSpec((HID, DH), lambda i: (0, 0)),
    ],
    out_specs=(pl.BlockSpec((RB, DH), lambda i: (i, 0)),
               pl.BlockSpec((RB, DH), lambda i: (i, 0))),
    out_shape=(jax.ShapeDtypeStruct((N, DH), jnp.bfloat16),
               jax.ShapeDtypeStruct((N, DH), jnp.bfloat16)),
)


def _add_body(qa0_ref, qb0_ref, qa1_ref, qb1_ref, b2_ref, out_ref):
    qa = qa0_ref[...] + qa1_ref[...]
    qb = qb0_ref[...] + qb1_ref[...]
    out_ref[...] = jnp.concatenate([qa, qb], axis=1) + b2_ref[...]


_add = pl.pallas_call(
    _add_body,
    grid=(N // RB,),
    in_specs=[
        pl.BlockSpec((RB, DH), lambda i: (i, 0)),
        pl.BlockSpec((RB, DH), lambda i: (i, 0)),
        pl.BlockSpec((RB, DH), lambda i: (i, 0)),
        pl.BlockSpec((RB, DH), lambda i: (i, 0)),
        pl.BlockSpec((1, D), lambda i: (0, 0)),
    ],
    out_specs=pl.BlockSpec((RB, D), lambda i: (i, 0)),
    out_shape=jax.ShapeDtypeStruct((N, D), jnp.float32),
)


def kernel(x, edge_index, edge_attr, W1, b1, W2, b2):
    pad = E_PAD - E_ALL
    ar = jnp.arange(N, dtype=jnp.int32)
    pi = jnp.arange(pad, dtype=jnp.int32) % N
    row_f = jnp.concatenate([edge_index[0], ar, pi])
    col_f = jnp.concatenate([edge_index[1], ar, pi])
    ew_f = jnp.concatenate(
        [edge_attr, jnp.ones((N,), jnp.float32), jnp.zeros((pad,), jnp.float32)]
    )
    row3 = row_f.reshape(NT, NCHT, CH)
    col3 = col_f.reshape(NT, NCHT, CH)
    ew3 = ew_f.reshape(NT, NCHT, CH)

    x0 = x[:, :DH].astype(jnp.bfloat16)
    x1 = x[:, DH:].astype(jnp.bfloat16)
    zz = jnp.zeros((N, DH), jnp.float32)

    perm = []
    for blk in range(DH // 32):
        perm += [blk * 32 + 2 * i for i in range(16)]
        perm += [blk * 32 + 2 * i + 1 for i in range(16)]
    perm = jnp.asarray(perm, jnp.int32)
    inv = jnp.argsort(perm)

    norm = _norm_kernel(row3, col3, ew3)
    nbits = lax.bitcast_convert_type(norm, jnp.int32)
    packed = jnp.stack(
        [row_f.reshape(NT, NGRP, G, CHP),
         col_f.reshape(NT, NGRP, G, CHP),
         nbits.reshape(NT, NGRP, G, CHP)], axis=2)

    pa0, pb0, pa1, pb1 = _prop_kernel(x0, x1, packed, zz)
    hwa, hwb = _mlp(pa0, pb0, pa1, pb1,
                    W1[:DH][perm], W1[DH:][perm], b1.reshape(1, HID),
                    W2[:, :DH][:, inv], W2[:, DH:][:, inv])
    qa0, qb0, qa1, qb1 = _prop_kernel(hwa, hwb, packed, zz)
    return _add(qa0, qb0, qa1, qb1, b2.reshape(1, D))


# speedup vs baseline: 1.2880x; 1.2235x over previous
import functools

import jax
import jax.numpy as jnp
from jax import lax
from jax.experimental import pallas as pl
from jax.experimental.pallas import tpu as pltpu, tpu_sc as plsc

N = 10000
E0 = 320000
D = 128
DH = D // 2
HID = 1024

NC = 2
NS = 16
NT = NC * NS
CH = 128

E_ALL = E0 + N
NCHT = (-(-E_ALL // (NT * CH)) + 7) // 8 * 8
EPT = NCHT * CH
E_PAD = EPT * NT
GN = 8
NGN = NCHT // GN

CHP = 64
NCHP = EPT // CHP
G = 16
NGRP = NCHP // G
GE = G * CHP

DEG_PAD = 10240
DPT = DEG_PAD // NS

RPT = 624
TAIL = N - NS * RPT

_mesh = plsc.VectorSubcoreMesh(core_axis_name="c", subcore_axis_name="s")
_sc_params = pltpu.CompilerParams(needs_layout_passes=False, use_tc_tiling_on_sc=False)


@functools.partial(
    pl.kernel,
    mesh=_mesh,
    compiler_params=_sc_params,
    out_type=jax.ShapeDtypeStruct((E_PAD,), jnp.float32),
    scratch_types=[
        pltpu.VMEM_SHARED((DEG_PAD,), jnp.float32),
        pltpu.VMEM_SHARED((DEG_PAD,), jnp.float32),
        pltpu.VMEM((GN, CH), jnp.int32),
        pltpu.VMEM((GN, CH), jnp.float32),
        pltpu.VMEM((GN, CH), jnp.int32),
        pltpu.VMEM((DEG_PAD,), jnp.float32),
        pltpu.VMEM((GN * CH,), jnp.float32),
        pltpu.SemaphoreType.DMA,
        pltpu.SemaphoreType.DMA,
    ],
)
def _norm_kernel(row3d, col3d, ew3d, norm_out,
                 deg_s, dis_s, cbufc, ebufc, rbufc, dis_v, normc, sma, smb):
    c = lax.axis_index("c")
    s = lax.axis_index("s")
    w = 2 * s + c

    z16 = jnp.zeros((16,), jnp.float32)

    def zb(j, carry):
        normc[pl.ds(j * 16, 16)] = z16
        return carry
    lax.fori_loop(0, DPT // 16, zb, 0)
    pltpu.sync_copy(normc.at[pl.ds(0, DPT)], deg_s.at[pl.ds(s * DPT, DPT)])
    plsc.subcore_barrier()

    for i2 in range(2):
        def dgrp(grp, carry, i2=i2):
            da = pltpu.async_copy(
                col3d.at[2 * s + i2].at[pl.ds(grp * GN, GN)], cbufc, sma)
            db = pltpu.async_copy(
                ew3d.at[2 * s + i2].at[pl.ds(grp * GN, GN)], ebufc, smb)
            da.wait()
            db.wait()
            scs = [pltpu.async_copy(ebufc.at[j], deg_s.at[cbufc.at[j]], sma,
                                    add=True) for j in range(GN)]
            for d in scs:
                d.wait()
            return carry
        lax.fori_loop(0, NGN, dgrp, 0)
    plsc.subcore_barrier()

    pltpu.sync_copy(deg_s.at[pl.ds(s * DPT, DPT)], dis_v.at[pl.ds(0, DPT)])

    def rsq(k, carry):
        x = dis_v[pl.ds(k * 16, 16)]
        i = lax.bitcast_convert_type(x, jnp.int32)
        i = 0x5F3759DF - lax.shift_right_arithmetic(i, 1)
        y = lax.bitcast_convert_type(i, jnp.float32)
        for _ in range(3):
            y = y * (1.5 - 0.5 * x * y * y)
        y = jnp.where(x > 0.0, y, 0.0)
        dis_v[pl.ds(k * 16, 16)] = y
        return carry
    lax.fori_loop(0, DPT // 16, rsq, 0)
    pltpu.sync_copy(dis_v.at[pl.ds(0, DPT)], dis_s.at[pl.ds(s * DPT, DPT)])
    plsc.subcore_barrier()
    pltpu.sync_copy(dis_s, dis_v)

    def ngrp(grp, carry):
        pltpu.sync_copy(row3d.at[w].at[pl.ds(grp * GN, GN)], rbufc)
        pltpu.sync_copy(col3d.at[w].at[pl.ds(grp * GN, GN)], cbufc)
        pltpu.sync_copy(ew3d.at[w].at[pl.ds(grp * GN, GN)], ebufc)

        def nrm(j, c2):
            for q in range(8):
                r16 = rbufc[j, pl.ds(q * 16, 16)]
                c16 = cbufc[j, pl.ds(q * 16, 16)]
                e16 = ebufc[j, pl.ds(q * 16, 16)]
                dr = plsc.load_gather(dis_v, [r16])
                dc = plsc.load_gather(dis_v, [c16])
                normc[pl.ds(j * CH + q * 16, 16)] = dr * e16 * dc
            return c2
        lax.fori_loop(0, GN, nrm, 0)
        pltpu.sync_copy(normc,
                        norm_out.at[pl.ds(w * EPT + grp * (GN * CH), GN * CH)])
        return carry
    lax.fori_loop(0, NGN, ngrp, 0)


@functools.partial(
    pl.kernel,
    mesh=_mesh,
    compiler_params=_sc_params,
    out_type=(jax.ShapeDtypeStruct((N, DH), jnp.float32),
              jax.ShapeDtypeStruct((N, DH), jnp.float32),
              jax.ShapeDtypeStruct((N, DH), jnp.float32),
              jax.ShapeDtypeStruct((N, DH), jnp.float32)),
    scratch_types=[
        pltpu.VMEM_SHARED((N, DH), jnp.float32),
        pltpu.VMEM((3, G, CHP), jnp.int32),
        pltpu.VMEM((3, CHP, DH), jnp.float32),
        pltpu.SemaphoreType.DMA,
        pltpu.SemaphoreType.DMA,
        pltpu.SemaphoreType.DMA,
        pltpu.SemaphoreType.DMA,
        pltpu.SemaphoreType.DMA,
        pltpu.SemaphoreType.DMA,
    ],
)
def _prop_kernel(z0, z1, packed, zz, pa0, pb0, pa1, pb1,
                 acc_s, pbuf, gbuf,
                 sg0, sg1, sg2, ss0, ss1, ss2):
    c = lax.axis_index("c")
    s = lax.axis_index("s")
    w = 2 * s + c
    base_s = s * RPT

    for p in range(2):
        zp = z0 if p == 0 else z1
        dst0, dst1 = (pa0, pa1) if p == 0 else (pb0, pb1)

        pltpu.sync_copy(zz.at[pl.ds(base_s, RPT)], acc_s.at[pl.ds(base_s, RPT)])

        @pl.when(s == NS - 1)
        def _():
            pltpu.sync_copy(zz.at[pl.ds(NS * RPT, TAIL)],
                            acc_s.at[pl.ds(NS * RPT, TAIL)])
        plsc.subcore_barrier()

        sgs = (sg0, sg1, sg2)
        sss = (ss0, ss1, ss2)

        def grp_body(grp, carry):
            pltpu.sync_copy(packed.at[w].at[grp], pbuf)

            gd = [None, None, None]
            sd = [None, None, None]
            gd[0] = pltpu.async_copy(zp.at[pbuf.at[0, 0]], gbuf.at[0], sgs[0])
            gd[1] = pltpu.async_copy(zp.at[pbuf.at[0, 1]], gbuf.at[1], sgs[1])
            for k in range(G):
                b = k % 3
                gd[b].wait()

                def scale(g, c3, k=k, b=b):
                    nbits = pbuf[2, k, pl.ds(g * 16, 16)]
                    n16 = lax.bitcast_convert_type(nbits, jnp.float32)
                    for bb16 in range(16):
                        nv = n16[bb16]
                        bb = g * 16 + bb16
                        for q in range(DH // 16):
                            gv = gbuf[b, bb, pl.ds(q * 16, 16)]
                            gbuf[b, bb, pl.ds(q * 16, 16)] = gv * nv
                    return c3
                lax.fori_loop(0, CHP // 16, scale, 0)
                sd[b] = pltpu.async_copy(gbuf.at[b], acc_s.at[pbuf.at[1, k]],
                                         sss[b], add=True)
                if k + 2 < G:
                    nb = (k + 2) % 3
                    if sd[nb] is not None:
                        sd[nb].wait()
                        sd[nb] = None
                    gd[nb] = pltpu.async_copy(zp.at[pbuf.at[0, k + 2]],
                                              gbuf.at[nb], sgs[nb])
            for b in range(3):
                if sd[b] is not None:
                    sd[b].wait()
            return carry
        lax.fori_loop(0, NGRP, grp_body, 0)
        plsc.subcore_barrier()

        @pl.when(c == 0)
        def _(dst=dst0):
            pltpu.sync_copy(acc_s.at[pl.ds(base_s, RPT)],
                            dst.at[pl.ds(base_s, RPT)])

            @pl.when(s == NS - 1)
            def _():
                pltpu.sync_copy(acc_s.at[pl.ds(NS * RPT, TAIL)],
                                dst.at[pl.ds(NS * RPT, TAIL)])

        @pl.when(c == 1)
        def _(dst=dst1):
            pltpu.sync_copy(acc_s.at[pl.ds(base_s, RPT)],
                            dst.at[pl.ds(base_s, RPT)])

            @pl.when(s == NS - 1)
            def _():
                pltpu.sync_copy(acc_s.at[pl.ds(NS * RPT, TAIL)],
                                dst.at[pl.ds(NS * RPT, TAIL)])
        plsc.subcore_barrier()


RB = 1000


def _mlp_body(pa0_ref, pb0_ref, pa1_ref, pb1_ref,
              w1a_ref, w1b_ref, b1_ref, w2a_ref, w2b_ref,
              hwa_ref, hwb_ref):
    pa = pa0_ref[...] + pa1_ref[...]
    pb = pb0_ref[...] + pb1_ref[...]
    h = jnp.dot(pa, w1a_ref[...], preferred_element_type=jnp.float32)
    h = h + jnp.dot(pb, w1b_ref[...], preferred_element_type=jnp.float32)
    h = jnp.maximum(h + b1_ref[...], 0.0)
    hwa_ref[...] = jnp.dot(h, w2a_ref[...], preferred_element_type=jnp.float32)
    hwb_ref[...] = jnp.dot(h, w2b_ref[...], preferred_element_type=jnp.float32)


_mlp = pl.pallas_call(
    _mlp_body,
    grid=(N // RB,),
    in_specs=[
        pl.BlockSpec((RB, DH), lambda i: (i, 0)),
        pl.BlockSpec((RB, DH), lambda i: (i, 0)),
        pl.BlockSpec((RB, DH), lambda i: (i, 0)),
        pl.BlockSpec((RB, DH), lambda i: (i, 0)),
        pl.BlockSpec((DH, HID), lambda i: (0, 0)),
        pl.BlockSpec((DH, HID), lambda i: (0, 0)),
        pl.BlockSpec((1, HID), lambda i: (0, 0)),
        pl.BlockSpec((HID, DH), lambda i: (0, 0)),
        pl.BlockSpec((HID, DH), lambda i: (0, 0)),
    ],
    out_specs=(pl.BlockSpec((RB, DH), lambda i: (i, 0)),
               pl.BlockSpec((RB, DH), lambda i: (i, 0))),
    out_shape=(jax.ShapeDtypeStruct((N, DH), jnp.float32),
               jax.ShapeDtypeStruct((N, DH), jnp.float32)),
)


def _add_body(qa0_ref, qb0_ref, qa1_ref, qb1_ref, b2_ref, out_ref):
    qa = qa0_ref[...] + qa1_ref[...]
    qb = qb0_ref[...] + qb1_ref[...]
    out_ref[...] = jnp.concatenate([qa, qb], axis=1) + b2_ref[...]


_add = pl.pallas_call(
    _add_body,
    grid=(N // RB,),
    in_specs=[
        pl.BlockSpec((RB, DH), lambda i: (i, 0)),
        pl.BlockSpec((RB, DH), lambda i: (i, 0)),
        pl.BlockSpec((RB, DH), lambda i: (i, 0)),
        pl.BlockSpec((RB, DH), lambda i: (i, 0)),
        pl.BlockSpec((1, D), lambda i: (0, 0)),
    ],
    out_specs=pl.BlockSpec((RB, D), lambda i: (i, 0)),
    out_shape=jax.ShapeDtypeStruct((N, D), jnp.float32),
)


def kernel(x, edge_index, edge_attr, W1, b1, W2, b2):
    pad = E_PAD - E_ALL
    ar = jnp.arange(N, dtype=jnp.int32)
    pi = jnp.arange(pad, dtype=jnp.int32) % N
    row_f = jnp.concatenate([edge_index[0], ar, pi])
    col_f = jnp.concatenate([edge_index[1], ar, pi])
    ew_f = jnp.concatenate(
        [edge_attr, jnp.ones((N,), jnp.float32), jnp.zeros((pad,), jnp.float32)]
    )
    row3 = row_f.reshape(NT, NCHT, CH)
    col3 = col_f.reshape(NT, NCHT, CH)
    ew3 = ew_f.reshape(NT, NCHT, CH)

    x0 = x[:, :DH]
    x1 = x[:, DH:]
    zz = jnp.zeros((N, DH), jnp.float32)

    norm = _norm_kernel(row3, col3, ew3)
    nbits = lax.bitcast_convert_type(norm, jnp.int32)
    packed = jnp.stack(
        [row_f.reshape(NT, NGRP, G, CHP),
         col_f.reshape(NT, NGRP, G, CHP),
         nbits.reshape(NT, NGRP, G, CHP)], axis=2)

    pa0, pb0, pa1, pb1 = _prop_kernel(x0, x1, packed, zz)
    hwa, hwb = _mlp(pa0, pb0, pa1, pb1,
                    W1[:DH], W1[DH:], b1.reshape(1, HID),
                    W2[:, :DH], W2[:, DH:])
    qa0, qb0, qa1, qb1 = _prop_kernel(hwa, hwb, packed, zz)
    return _add(qa0, qb0, qa1, qb1, b2.reshape(1, D))
